# baseline (speedup 1.0000x reference)
import jax
import jax.numpy as jnp
import numpy as _np
from jax.experimental import pallas as pl

H = 1024
VD = 384
ID = 2048
NL = 12
K = 8192
CD = 8
B = 2
TT = 2048
TH = TT // 2

_F32 = jnp.float32
_BF = jnp.bfloat16


def _mm_t(a, w):
    return jax.lax.dot_general(a.astype(_BF), w.astype(_BF),
                               (((1,), (1,)), ((), ())),
                               preferred_element_type=_F32)


def _mm(a, w):
    return jax.lax.dot_general(a.astype(_BF), w.astype(_BF),
                               (((1,), (0,)), ((), ())),
                               preferred_element_type=_F32)


def _mm_exact(a, w):
    return jax.lax.dot_general(a, w, (((1,), (0,)), ((), ())),
                               preferred_element_type=_F32,
                               precision=jax.lax.Precision.HIGHEST)


def _shift(x, off, n):
    c = x.shape[1]
    if off == 0:
        return x
    if off < 0:
        return jnp.concatenate(
            [jnp.zeros((-off, c), _F32), x[: n + off]], axis=0)
    return jnp.concatenate([x[off:], jnp.zeros((off, c), _F32)], axis=0)


_C_ERF = [_np.float32(v) for v in
          [7.85386146e-05, -0.000801019371, 0.00518832775, -0.0268538129,
           0.112835854, -0.37612626, 1.12837911]]
_C_PA = [_np.float32(v) for v in
         [0.0232682, -0.138703942, 0.368742466, -0.582473278, 0.621000469,
          -0.494451523, 0.340488, -0.274112701, 0.563825965]]
_C_PB = [_np.float32(v) for v in
         [-10.477664, 12.9772, -7.49551868, 2.92101908, -1.01526523,
          0.42184633, -0.282076746, 0.564189494]]


def _horner(t, coeffs):
    p = t * coeffs[0]
    p = p + coeffs[1]
    for c in coeffs[2:]:
        p = p * t
        p = p + c
    return p


def _erfc(v):
    one = _np.float32(1.0)
    ax = jnp.abs(v)
    t = v * v
    small = one - (v * _horner(t, _C_ERF))
    nt = -t
    y = jnp.exp(nt) * (one / ax)
    w2 = one / t
    p = jnp.where(ax < _np.float32(2.0), _horner(w2, _C_PA), _horner(w2, _C_PB))
    r = y * p
    r = jnp.where(nt < _np.float32(-88.7228394), _np.float32(0.0), r)
    r = jnp.where(v < _np.float32(0.0), _np.float32(2.0) - r, r)
    return jnp.where(ax < one, small, r)


def _gelu(x):
    return (x * _np.float32(0.5)) * _erfc((-x) * _np.float32(0.707106769))


_RECIP_VD = _np.float32(1.0) / _np.float32(384.0)


def _sum384(v):
    s = v[:, :128] + v[:, 128:256]
    s = s + v[:, 256:]
    acc = s[:, 0:8]
    for g in range(1, 16):
        acc = acc + s[:, 8 * g:8 * g + 8]
    a = acc[:, 0:4] + acc[:, 4:8]
    b = a[:, 0:2] + a[:, 2:4]
    return b[:, 0:1] + b[:, 1:2]


def _ln(x, g, b):
    mu = _sum384(x) * _RECIP_VD
    c = x - mu
    var = _sum384(c * c) * _RECIP_VD
    return c / jnp.sqrt(var + _np.float32(1e-6)) * g + b


def _sum8_lanes(v):
    a = v[:, 0:4] + v[:, 4:8]
    b = a[:, 0:2] + a[:, 2:4]
    return b[:, 0:1] + b[:, 1:2]


def _sum8_rows(v):
    a = v[0:4, :] + v[4:8, :]
    b = a[0:2, :] + a[2:4, :]
    return b[0:1, :] + b[1:2, :]


def _down_kernel(xr_ref, wf_ref, b_ref, o_ref):
    even = xr_ref[0, :, 0, :]
    odd = xr_ref[0, :, 1, :]
    cols = jnp.concatenate([_shift(odd, -1, TH), even, odd], axis=1)
    acc = _mm_t(cols, wf_ref[...]) + b_ref[...]
    o_ref[0] = _gelu(acc)


def _downsample(x, wf, b):
    xr = x.reshape(B, TH, 2, H)
    return pl.pallas_call(
        _down_kernel,
        grid=(B,),
        in_specs=[
            pl.BlockSpec((1, TH, 2, H), lambda i: (i, 0, 0, 0)),
            pl.BlockSpec((H, 3 * H), lambda i: (0, 0)),
            pl.BlockSpec((1, H), lambda i: (0, 0)),
        ],
        out_specs=pl.BlockSpec((1, TH, H), lambda i: (i, 0, 0)),
        out_shape=jax.ShapeDtypeStruct((B, TH, H), _F32),
    )(xr, wf, b.reshape(1, H))


def _embed_kernel(x_ref, wf_ref, b_ref, g_ref, bn_ref, o_ref):
    xb = x_ref[0]
    cols = jnp.concatenate([_shift(xb, k - 3, TH) for k in range(7)], axis=1)
    acc = _mm_t(cols, wf_ref[...]) + b_ref[...]
    o_ref[0] = _ln(acc, g_ref[...], bn_ref[...])


def _embed_ln(x, wf, b, g, bn, cin):
    return pl.pallas_call(
        _embed_kernel,
        grid=(B,),
        in_specs=[
            pl.BlockSpec((1, TH, cin), lambda i: (i, 0, 0)),
            pl.BlockSpec((VD, 7 * cin), lambda i: (0, 0)),
            pl.BlockSpec((1, VD), lambda i: (0, 0)),
            pl.BlockSpec((1, VD), lambda i: (0, 0)),
            pl.BlockSpec((1, VD), lambda i: (0, 0)),
        ],
        out_specs=pl.BlockSpec((1, TH, VD), lambda i: (i, 0, 0)),
        out_shape=jax.ShapeDtypeStruct((B, TH, VD), _F32),
    )(x, wf, b.reshape(1, VD), g.reshape(1, VD), bn.reshape(1, VD))


def _dwconv(x, wv):
    xq = x.astype(_BF).astype(_F32)
    acc = _shift(xq, -3, TH) * wv[0][None, :]
    for k in range(1, 7):
        acc = acc + _shift(xq, k - 3, TH) * wv[k][None, :]
    return acc


def _block_kernel(x_ref, dw_ref, dwb_ref, g_ref, bn_ref, w1_ref, b1_ref,
                  w2_ref, b2_ref, gam_ref, o_ref):
    xb = x_ref[0]
    h = _dwconv(xb, dw_ref[...]) + dwb_ref[...]
    h = _ln(h, g_ref[...], bn_ref[...])
    h = _gelu(_mm_t(h, w1_ref[...]) + b1_ref[...])
    h = _mm_t(h, w2_ref[...]) + b2_ref[...]
    o_ref[0] = xb + gam_ref[...] * h


def _convnext_block(x, blk):
    dwt = blk['dw_w'][:, 0, :].T
    return pl.pallas_call(
        _block_kernel,
        grid=(B,),
        in_specs=[
            pl.BlockSpec((1, TH, VD), lambda i: (i, 0, 0)),
            pl.BlockSpec((7, VD), lambda i: (0, 0)),
            pl.BlockSpec((1, VD), lambda i: (0, 0)),
            pl.BlockSpec((1, VD), lambda i: (0, 0)),
            pl.BlockSpec((1, VD), lambda i: (0, 0)),
            pl.BlockSpec((ID, VD), lambda i: (0, 0)),
            pl.BlockSpec((1, ID), lambda i: (0, 0)),
            pl.BlockSpec((VD, ID), lambda i: (0, 0)),
            pl.BlockSpec((1, VD), lambda i: (0, 0)),
            pl.BlockSpec((1, VD), lambda i: (0, 0)),
        ],
        out_specs=pl.BlockSpec((1, TH, VD), lambda i: (i, 0, 0)),
        out_shape=jax.ShapeDtypeStruct((B, TH, VD), _F32),
    )(x, dwt, blk['dw_b'].reshape(1, VD), blk['n_g'].reshape(1, VD),
      blk['n_b'].reshape(1, VD), blk['pw1_w'], blk['pw1_b'].reshape(1, ID),
      blk['pw2_w'], blk['pw2_b'].reshape(1, VD), blk['gamma'].reshape(1, VD))


def _lnlin_kernel(x_ref, g_ref, bn_ref, w_ref, b_ref, o_ref):
    y = _ln(x_ref[0], g_ref[...], bn_ref[...])
    o_ref[0] = _mm_t(y, w_ref[...]) + b_ref[...]


def _ln_linear(x, g, bn, w, b):
    return pl.pallas_call(
        _lnlin_kernel,
        grid=(B,),
        in_specs=[
            pl.BlockSpec((1, TH, VD), lambda i: (i, 0, 0)),
            pl.BlockSpec((1, VD), lambda i: (0, 0)),
            pl.BlockSpec((1, VD), lambda i: (0, 0)),
            pl.BlockSpec((H, VD), lambda i: (0, 0)),
            pl.BlockSpec((1, H), lambda i: (0, 0)),
        ],
        out_specs=pl.BlockSpec((1, TH, H), lambda i: (i, 0, 0)),
        out_shape=jax.ShapeDtypeStruct((B, TH, H), _F32),
    )(x, g.reshape(1, VD), bn.reshape(1, VD), w, b.reshape(1, H))


_VQ_TILE = 256
_VQ_GRID = (B * TH) // _VQ_TILE


def _vq_kernel(e_ref, inw_ref, inb_ref, cb_ref, cbt_ref, outw_ref, outb_ref,
               zq_ref, idx_ref, sum_ref):
    i = pl.program_id(0)
    ze = _mm_t(e_ref[...], inw_ref[...]) + inb_ref[...]
    n = jnp.sqrt(_sum8_lanes(ze * ze))
    zn = ze / jnp.maximum(n, _np.float32(1e-12))
    cbt = cbt_ref[...]
    cn = jnp.sqrt(_sum8_rows(cbt * cbt))
    cbnt = cbt / jnp.maximum(cn, _np.float32(1e-12))
    sim = _mm(zn, cbnt)
    dist = (_sum8_lanes(zn * zn) - 2.0 * sim
            + _sum8_rows(cbnt * cbnt))
    m = jnp.min(dist, axis=-1, keepdims=True)
    iota = jax.lax.broadcasted_iota(jnp.int32, (_VQ_TILE, K), 1)
    idx = jnp.min(jnp.where(dist <= m, iota, K), axis=-1)
    oh = (iota == idx[:, None]).astype(_F32)
    zq = _mm_exact(oh, cb_ref[...])
    d = ze - zq
    s = jnp.sum(d * d)

    @pl.when(i == 0)
    def _():
        sum_ref[...] = jnp.zeros_like(sum_ref)

    sum_ref[...] = sum_ref[...] + s
    idx_ref[0, 0, :] = idx
    zq_st = ze + (zq - ze)
    zq_ref[...] = _mm_t(zq_st, outw_ref[...]) + outb_ref[...]


def _vq2_kernel(ze_ref, idx_ref, cb_ref, outw_ref, outb_ref,
                zq_ref, sum_ref):
    i = pl.program_id(0)
    ze = ze_ref[...]
    idx = idx_ref[0, 0, :]
    iota = jax.lax.broadcasted_iota(jnp.int32, (_VQ_TILE, K), 1)
    oh = (iota == idx[:, None]).astype(_F32)
    zq = _mm_exact(oh, cb_ref[...])
    d = ze - zq
    s = jnp.sum(d * d)

    @pl.when(i == 0)
    def _():
        sum_ref[...] = jnp.zeros_like(sum_ref)

    sum_ref[...] = sum_ref[...] + s
    zq_st = ze + (zq - ze)
    zq_ref[...] = _mm_t(zq_st, outw_ref[...]) + outb_ref[...]


def _vq_gather(ze_flat, idx_flat, vqp):
    outw = vqp['out_w'][:, :, 0]
    cb = vqp['codebook']
    zq, s = pl.pallas_call(
        _vq2_kernel,
        grid=(_VQ_GRID,),
        in_specs=[
            pl.BlockSpec((_VQ_TILE, CD), lambda i: (i, 0)),
            pl.BlockSpec((1, 1, _VQ_TILE), lambda i: (i, 0, 0)),
            pl.BlockSpec((K, CD), lambda i: (0, 0)),
            pl.BlockSpec((H, CD), lambda i: (0, 0)),
            pl.BlockSpec((1, H), lambda i: (0, 0)),
        ],
        out_specs=[
            pl.BlockSpec((_VQ_TILE, H), lambda i: (i, 0)),
            pl.BlockSpec((1, 1), lambda i: (0, 0)),
        ],
        out_shape=[
            jax.ShapeDtypeStruct((B * TH, H), _F32),
            jax.ShapeDtypeStruct((1, 1), _F32),
        ],
    )(ze_flat, idx_flat.reshape(_VQ_GRID, 1, _VQ_TILE), cb, outw,
      vqp['out_b'].reshape(1, H))
    loss = (1.0 + 0.15) * s[0, 0] / (B * CD * TH)
    return zq, loss


def _normalize_ref(v, eps=1e-12):
    n = jnp.sqrt(jnp.sum(v * v, axis=-1, keepdims=True))
    return v / jnp.maximum(n, eps)


def _vq(e_flat, vqp):
    inw = vqp['in_w'][:, :, 0]
    outw = vqp['out_w'][:, :, 0]
    cb = vqp['codebook']
    cbt = cb.T
    zq, idx, s = pl.pallas_call(
        _vq_kernel,
        grid=(_VQ_GRID,),
        in_specs=[
            pl.BlockSpec((_VQ_TILE, H), lambda i: (i, 0)),
            pl.BlockSpec((CD, H), lambda i: (0, 0)),
            pl.BlockSpec((1, CD), lambda i: (0, 0)),
            pl.BlockSpec((K, CD), lambda i: (0, 0)),
            pl.BlockSpec((CD, K), lambda i: (0, 0)),
            pl.BlockSpec((H, CD), lambda i: (0, 0)),
            pl.BlockSpec((1, H), lambda i: (0, 0)),
        ],
        out_specs=[
            pl.BlockSpec((_VQ_TILE, H), lambda i: (i, 0)),
            pl.BlockSpec((1, 1, _VQ_TILE), lambda i: (i, 0, 0)),
            pl.BlockSpec((1, 1), lambda i: (0, 0)),
        ],
        out_shape=[
            jax.ShapeDtypeStruct((B * TH, H), _F32),
            jax.ShapeDtypeStruct((_VQ_GRID, 1, _VQ_TILE), jnp.int32),
            jax.ShapeDtypeStruct((1, 1), _F32),
        ],
    )(e_flat, inw, vqp['in_b'].reshape(1, CD), cb, cbt, outw,
      vqp['out_b'].reshape(1, H))
    indices = idx.reshape(B, TH)
    loss = (1.0 + 0.15) * s[0, 0] / (B * CD * TH)
    return zq, indices, loss


def _up_kernel(d_ref, w_ref, b_ref, o_ref):
    db = d_ref[0]
    d_m1 = _shift(db, -1, TH)
    d_p1 = _shift(db, 1, TH)
    bias = b_ref[...]
    even = _mm_t(d_m1, w_ref[0]) + _mm_t(db, w_ref[1]) + bias
    odd = _mm_t(db, w_ref[2]) + _mm_t(d_p1, w_ref[3]) + bias
    o_ref[0, :, 0, :] = even
    o_ref[0, :, 1, :] = odd


def _upsample(d, w, b):
    out = pl.pallas_call(
        _up_kernel,
        grid=(B,),
        in_specs=[
            pl.BlockSpec((1, TH, H), lambda i: (i, 0, 0)),
            pl.BlockSpec((4, H, H), lambda i: (0, 0, 0)),
            pl.BlockSpec((1, H), lambda i: (0, 0)),
        ],
        out_specs=pl.BlockSpec((1, TH, 2, H), lambda i: (i, 0, 0, 0)),
        out_shape=jax.ShapeDtypeStruct((B, TH, 2, H), _F32),
    )(d, w, b.reshape(1, H))
    return out.reshape(B, TT, H)


def _backbone(tok, p, cin):
    ewf = jnp.moveaxis(p['embed_w'], 2, 1).reshape(VD, 7 * cin)
    y = _embed_ln(tok, ewf, p['embed_b'], p['norm_g'], p['norm_b'], cin)
    for blk in p['blocks']:
        y = _convnext_block(y, blk)
    return y


def _pw_kernel(y_ref, res_ref, w1_ref, b1_ref, w2_ref, b2_ref, gam_ref,
               o_ref):
    h = _gelu(_mm_t(y_ref[0], w1_ref[...]) + b1_ref[...])
    h = _mm_t(h, w2_ref[...]) + b2_ref[...]
    o_ref[0] = res_ref[0] + gam_ref[...] * h


def _pw_chain(y, res, blk):
    return pl.pallas_call(
        _pw_kernel,
        grid=(B,),
        in_specs=[
            pl.BlockSpec((1, TH, VD), lambda i: (i, 0, 0)),
            pl.BlockSpec((1, TH, VD), lambda i: (i, 0, 0)),
            pl.BlockSpec((ID, VD), lambda i: (0, 0)),
            pl.BlockSpec((1, ID), lambda i: (0, 0)),
            pl.BlockSpec((VD, ID), lambda i: (0, 0)),
            pl.BlockSpec((1, VD), lambda i: (0, 0)),
            pl.BlockSpec((1, VD), lambda i: (0, 0)),
        ],
        out_specs=pl.BlockSpec((1, TH, VD), lambda i: (i, 0, 0)),
        out_shape=jax.ShapeDtypeStruct((B, TH, VD), _F32),
    )(y, res, blk['pw1_w'], blk['pw1_b'].reshape(1, ID), blk['pw2_w'],
      blk['pw2_b'].reshape(1, VD), blk['gamma'].reshape(1, VD))


def _ln_ref(x, g, b, eps=1e-6):
    mu = jnp.mean(x, axis=-1, keepdims=True)
    var = jnp.mean((x - mu) ** 2, axis=-1, keepdims=True)
    return (x - mu) / jnp.sqrt(var + eps) * g + b


def _enc_backbone(tok, p):
    t = jnp.transpose(tok, (0, 2, 1))
    t = jax.lax.conv_general_dilated(
        t, p['embed_w'], window_strides=(1,), padding=[(3, 3)],
        dimension_numbers=('NCH', 'OIH', 'NCH')) + p['embed_b'][None, :, None]
    y = _ln_ref(jnp.transpose(t, (0, 2, 1)), p['norm_g'], p['norm_b'])
    for blk in p['blocks']:
        t = jnp.transpose(y, (0, 2, 1))
        t = jax.lax.conv_general_dilated(
            t, blk['dw_w'], window_strides=(1,), padding=[(3, 3)],
            dimension_numbers=('NCH', 'OIH', 'NCH'),
            feature_group_count=VD) + blk['dw_b'][None, :, None]
        h = _ln_ref(jnp.transpose(t, (0, 2, 1)), blk['n_g'], blk['n_b'])
        y = _pw_chain(h, y, blk)
    return _ln_ref(y, p['final_g'], p['final_b'])


def _lin_kernel(x_ref, w_ref, b_ref, o_ref):
    o_ref[0] = _mm_t(x_ref[0], w_ref[...]) + b_ref[...]


def _linear(x, w, b):
    return pl.pallas_call(
        _lin_kernel,
        grid=(B,),
        in_specs=[
            pl.BlockSpec((1, TH, VD), lambda i: (i, 0, 0)),
            pl.BlockSpec((H, VD), lambda i: (0, 0)),
            pl.BlockSpec((1, H), lambda i: (0, 0)),
        ],
        out_specs=pl.BlockSpec((1, TH, H), lambda i: (i, 0, 0)),
        out_shape=jax.ShapeDtypeStruct((B, TH, H), _F32),
    )(x, w, b.reshape(1, H))


def kernel(x, params):
    hh = jnp.transpose(x, (0, 2, 1))
    hh = jax.lax.conv_general_dilated(
        hh, params['down_w'], window_strides=(2,), padding=[(1, 1)],
        dimension_numbers=('NCH', 'OIH', 'NCH')) + params['down_b'][None, :, None]
    h = jnp.transpose(jax.nn.gelu(hh, approximate=False), (0, 2, 1))

    p = params['enc']
    t = jnp.transpose(h, (0, 2, 1))
    t = jax.lax.conv_general_dilated(
        t, p['embed_w'], window_strides=(1,), padding=[(3, 3)],
        dimension_numbers=('NCH', 'OIH', 'NCH')) + p['embed_b'][None, :, None]
    y = _ln_ref(jnp.transpose(t, (0, 2, 1)), p['norm_g'], p['norm_b'])
    for blk in p['blocks']:
        t = jnp.transpose(y, (0, 2, 1))
        t = jax.lax.conv_general_dilated(
            t, blk['dw_w'], window_strides=(1,), padding=[(3, 3)],
            dimension_numbers=('NCH', 'OIH', 'NCH'),
            feature_group_count=VD) + blk['dw_b'][None, :, None]
        hb = _ln_ref(jnp.transpose(t, (0, 2, 1)), blk['n_g'], blk['n_b'])
        hb = jax.nn.gelu(hb @ blk['pw1_w'].T + blk['pw1_b'], approximate=False)
        hb = hb @ blk['pw2_w'].T + blk['pw2_b']
        y = y + blk['gamma'] * hb
    eb = _ln_ref(y, p['final_g'], p['final_b'])
    e = eb @ params['enc_lin_w'].T + params['enc_lin_b']

    vqp = params['vq']
    ze = jax.lax.conv_general_dilated(
        jnp.transpose(e, (0, 2, 1)), vqp['in_w'], window_strides=(1,),
        padding=[(0, 0)], dimension_numbers=('NCH', 'OIH', 'NCH'))
    ze = ze + vqp['in_b'][None, :, None]
    ze_flat = jnp.transpose(ze, (0, 2, 1)).reshape(B * TH, CD)
    enc_n = _normalize_ref(ze_flat)
    cb_n = _normalize_ref(vqp['codebook'])
    dist = (jnp.sum(enc_n ** 2, axis=1, keepdims=True)
            - 2.0 * enc_n @ cb_n.T
            + jnp.sum(cb_n ** 2, axis=1)[None, :])
    idx_flat = jnp.argmax(-dist, axis=1)
    indices = idx_flat.reshape(B, TH)
    zq_flat, loss = _vq_gather(ze_flat, idx_flat, vqp)
    zq = zq_flat.reshape(B, TH, H)

    db = _backbone(zq, params['dec'], H)
    d = _ln_linear(db, params['dec']['final_g'], params['dec']['final_b'],
                   params['dec_lin_w'], params['dec_lin_b'])

    uw = jnp.moveaxis(params['up_w'], 2, 0)
    uws = jnp.stack([uw[0], uw[1] + uw[2], uw[0] + uw[1], uw[2]])
    x_rec = _upsample(d, uws, params['up_b'])

    all_indices = indices[None].astype(jnp.int32)
    return x_rec, loss, all_indices


# speedup vs baseline: 1.3222x; 1.3222x over previous
import jax
import jax.numpy as jnp
import numpy as _np
from jax.experimental import pallas as pl

H = 1024
VD = 384
ID = 2048
NL = 12
K = 8192
CD = 8
B = 2
TT = 2048
TH = TT // 2

_F32 = jnp.float32
_BF = jnp.bfloat16


def _mm_t(a, w):
    return jax.lax.dot_general(a.astype(_BF), w.astype(_BF),
                               (((1,), (1,)), ((), ())),
                               preferred_element_type=_F32)


def _mm_exact(a, w):
    return jax.lax.dot_general(a, w, (((1,), (0,)), ((), ())),
                               preferred_element_type=_F32,
                               precision=jax.lax.Precision.HIGHEST)


def _shift(x, off, n):
    c = x.shape[1]
    if off == 0:
        return x
    if off < 0:
        return jnp.concatenate(
            [jnp.zeros((-off, c), _F32), x[: n + off]], axis=0)
    return jnp.concatenate([x[off:], jnp.zeros((off, c), _F32)], axis=0)


def _gelu(x):
    return (x * _np.float32(0.5)) * (
        _np.float32(1.0) + jax.lax.erf(x * _np.float32(0.707106769)))


_RECIP_VD = _np.float32(1.0) / _np.float32(384.0)


def _sum384(v):
    s = v[:, :128] + v[:, 128:256]
    s = s + v[:, 256:]
    acc = s[:, 0:8]
    for g in range(1, 16):
        acc = acc + s[:, 8 * g:8 * g + 8]
    a = acc[:, 0:4] + acc[:, 4:8]
    b = a[:, 0:2] + a[:, 2:4]
    return b[:, 0:1] + b[:, 1:2]


def _ln(x, g, b):
    mu = _sum384(x) * _RECIP_VD
    c = x - mu
    var = _sum384(c * c) * _RECIP_VD
    return c / jnp.sqrt(var + _np.float32(1e-6)) * g + b


def _ln_ref(x, g, b, eps=1e-6):
    mu = jnp.mean(x, axis=-1, keepdims=True)
    var = jnp.mean((x - mu) ** 2, axis=-1, keepdims=True)
    return (x - mu) / jnp.sqrt(var + eps) * g + b


def _normalize_ref(v, eps=1e-12):
    n = jnp.sqrt(jnp.sum(v * v, axis=-1, keepdims=True))
    return v / jnp.maximum(n, eps)


def _embed_kernel(x_ref, wf_ref, b_ref, g_ref, bn_ref, o_ref):
    xb = x_ref[0]
    cols = jnp.concatenate([_shift(xb, k - 3, TH) for k in range(7)], axis=1)
    acc = _mm_t(cols, wf_ref[...]) + b_ref[...]
    o_ref[0] = _ln(acc, g_ref[...], bn_ref[...])


def _embed_ln(x, wf, b, g, bn, cin):
    return pl.pallas_call(
        _embed_kernel,
        grid=(B,),
        in_specs=[
            pl.BlockSpec((1, TH, cin), lambda i: (i, 0, 0)),
            pl.BlockSpec((VD, 7 * cin), lambda i: (0, 0)),
            pl.BlockSpec((1, VD), lambda i: (0, 0)),
            pl.BlockSpec((1, VD), lambda i: (0, 0)),
            pl.BlockSpec((1, VD), lambda i: (0, 0)),
        ],
        out_specs=pl.BlockSpec((1, TH, VD), lambda i: (i, 0, 0)),
        out_shape=jax.ShapeDtypeStruct((B, TH, VD), _F32),
    )(x, wf, b.reshape(1, VD), g.reshape(1, VD), bn.reshape(1, VD))


def _dwconv(x, wv):
    xq = x.astype(_BF).astype(_F32)
    acc = _shift(xq, -3, TH) * wv[0][None, :]
    for k in range(1, 7):
        acc = acc + _shift(xq, k - 3, TH) * wv[k][None, :]
    return acc


def _block_kernel(x_ref, dw_ref, dwb_ref, g_ref, bn_ref, w1_ref, b1_ref,
                  w2_ref, b2_ref, gam_ref, o_ref):
    xb = x_ref[0]
    h = _dwconv(xb, dw_ref[...]) + dwb_ref[...]
    h = _ln(h, g_ref[...], bn_ref[...])
    h = _gelu(_mm_t(h, w1_ref[...]) + b1_ref[...])
    h = _mm_t(h, w2_ref[...]) + b2_ref[...]
    o_ref[0] = xb + gam_ref[...] * h


def _convnext_block(x, blk):
    dwt = blk['dw_w'][:, 0, :].T
    return pl.pallas_call(
        _block_kernel,
        grid=(B,),
        in_specs=[
            pl.BlockSpec((1, TH, VD), lambda i: (i, 0, 0)),
            pl.BlockSpec((7, VD), lambda i: (0, 0)),
            pl.BlockSpec((1, VD), lambda i: (0, 0)),
            pl.BlockSpec((1, VD), lambda i: (0, 0)),
            pl.BlockSpec((1, VD), lambda i: (0, 0)),
            pl.BlockSpec((ID, VD), lambda i: (0, 0)),
            pl.BlockSpec((1, ID), lambda i: (0, 0)),
            pl.BlockSpec((VD, ID), lambda i: (0, 0)),
            pl.BlockSpec((1, VD), lambda i: (0, 0)),
            pl.BlockSpec((1, VD), lambda i: (0, 0)),
        ],
        out_specs=pl.BlockSpec((1, TH, VD), lambda i: (i, 0, 0)),
        out_shape=jax.ShapeDtypeStruct((B, TH, VD), _F32),
    )(x, dwt, blk['dw_b'].reshape(1, VD), blk['n_g'].reshape(1, VD),
      blk['n_b'].reshape(1, VD), blk['pw1_w'], blk['pw1_b'].reshape(1, ID),
      blk['pw2_w'], blk['pw2_b'].reshape(1, VD), blk['gamma'].reshape(1, VD))


def _lnlin_kernel(x_ref, g_ref, bn_ref, w_ref, b_ref, o_ref):
    y = _ln(x_ref[0], g_ref[...], bn_ref[...])
    o_ref[0] = _mm_t(y, w_ref[...]) + b_ref[...]


def _ln_linear(x, g, bn, w, b):
    return pl.pallas_call(
        _lnlin_kernel,
        grid=(B,),
        in_specs=[
            pl.BlockSpec((1, TH, VD), lambda i: (i, 0, 0)),
            pl.BlockSpec((1, VD), lambda i: (0, 0)),
            pl.BlockSpec((1, VD), lambda i: (0, 0)),
            pl.BlockSpec((H, VD), lambda i: (0, 0)),
            pl.BlockSpec((1, H), lambda i: (0, 0)),
        ],
        out_specs=pl.BlockSpec((1, TH, H), lambda i: (i, 0, 0)),
        out_shape=jax.ShapeDtypeStruct((B, TH, H), _F32),
    )(x, g.reshape(1, VD), bn.reshape(1, VD), w, b.reshape(1, H))


_VQ_TILE = 256
_VQ_GRID = (B * TH) // _VQ_TILE


def _vq2_kernel(ze_ref, idx_ref, cb_ref, outw_ref, outb_ref,
                zq_ref, sum_ref):
    i = pl.program_id(0)
    ze = ze_ref[...]
    idx = idx_ref[0, 0, :]
    iota = jax.lax.broadcasted_iota(jnp.int32, (_VQ_TILE, K), 1)
    oh = (iota == idx[:, None]).astype(_F32)
    zq = _mm_exact(oh, cb_ref[...])
    d = ze - zq
    s = jnp.sum(d * d)

    @pl.when(i == 0)
    def _():
        sum_ref[...] = jnp.zeros_like(sum_ref)

    sum_ref[...] = sum_ref[...] + s
    zq_st = ze + (zq - ze)
    zq_ref[...] = _mm_t(zq_st, outw_ref[...]) + outb_ref[...]


def _vq_gather(ze_flat, idx_flat, vqp):
    outw = vqp['out_w'][:, :, 0]
    cb = vqp['codebook']
    zq, s = pl.pallas_call(
        _vq2_kernel,
        grid=(_VQ_GRID,),
        in_specs=[
            pl.BlockSpec((_VQ_TILE, CD), lambda i: (i, 0)),
            pl.BlockSpec((1, 1, _VQ_TILE), lambda i: (i, 0, 0)),
            pl.BlockSpec((K, CD), lambda i: (0, 0)),
            pl.BlockSpec((H, CD), lambda i: (0, 0)),
            pl.BlockSpec((1, H), lambda i: (0, 0)),
        ],
        out_specs=[
            pl.BlockSpec((_VQ_TILE, H), lambda i: (i, 0)),
            pl.BlockSpec((1, 1), lambda i: (0, 0)),
        ],
        out_shape=[
            jax.ShapeDtypeStruct((B * TH, H), _F32),
            jax.ShapeDtypeStruct((1, 1), _F32),
        ],
    )(ze_flat, idx_flat.reshape(_VQ_GRID, 1, _VQ_TILE), cb, outw,
      vqp['out_b'].reshape(1, H))
    loss = (1.0 + 0.15) * s[0, 0] / (B * CD * TH)
    return zq, loss


def _up_kernel(d_ref, w_ref, b_ref, o_ref):
    db = d_ref[0]
    d_m1 = _shift(db, -1, TH)
    d_p1 = _shift(db, 1, TH)
    bias = b_ref[...]
    even = _mm_t(d_m1, w_ref[0]) + _mm_t(db, w_ref[1]) + bias
    odd = _mm_t(db, w_ref[2]) + _mm_t(d_p1, w_ref[3]) + bias
    o_ref[0, :, 0, :] = even
    o_ref[0, :, 1, :] = odd


def _upsample(d, w, b):
    out = pl.pallas_call(
        _up_kernel,
        grid=(B,),
        in_specs=[
            pl.BlockSpec((1, TH, H), lambda i: (i, 0, 0)),
            pl.BlockSpec((4, H, H), lambda i: (0, 0, 0)),
            pl.BlockSpec((1, H), lambda i: (0, 0)),
        ],
        out_specs=pl.BlockSpec((1, TH, 2, H), lambda i: (i, 0, 0, 0)),
        out_shape=jax.ShapeDtypeStruct((B, TH, 2, H), _F32),
    )(d, w, b.reshape(1, H))
    return out.reshape(B, TT, H)


def _backbone(tok, p, cin):
    ewf = jnp.moveaxis(p['embed_w'], 2, 1).reshape(VD, 7 * cin)
    y = _embed_ln(tok, ewf, p['embed_b'], p['norm_g'], p['norm_b'], cin)
    for blk in p['blocks']:
        y = _convnext_block(y, blk)
    return y


def kernel(x, params):
    hh = jnp.transpose(x, (0, 2, 1))
    hh = jax.lax.conv_general_dilated(
        hh, params['down_w'], window_strides=(2,), padding=[(1, 1)],
        dimension_numbers=('NCH', 'OIH', 'NCH')) + params['down_b'][None, :, None]
    h = jnp.transpose(jax.nn.gelu(hh, approximate=False), (0, 2, 1))

    p = params['enc']
    t = jnp.transpose(h, (0, 2, 1))
    t = jax.lax.conv_general_dilated(
        t, p['embed_w'], window_strides=(1,), padding=[(3, 3)],
        dimension_numbers=('NCH', 'OIH', 'NCH')) + p['embed_b'][None, :, None]
    y = _ln_ref(jnp.transpose(t, (0, 2, 1)), p['norm_g'], p['norm_b'])
    for blk in p['blocks']:
        t = jnp.transpose(y, (0, 2, 1))
        t = jax.lax.conv_general_dilated(
            t, blk['dw_w'], window_strides=(1,), padding=[(3, 3)],
            dimension_numbers=('NCH', 'OIH', 'NCH'),
            feature_group_count=VD) + blk['dw_b'][None, :, None]
        hb = _ln_ref(jnp.transpose(t, (0, 2, 1)), blk['n_g'], blk['n_b'])
        hb = jax.nn.gelu(hb @ blk['pw1_w'].T + blk['pw1_b'], approximate=False)
        hb = hb @ blk['pw2_w'].T + blk['pw2_b']
        y = y + blk['gamma'] * hb
    eb = _ln_ref(y, p['final_g'], p['final_b'])
    e = eb @ params['enc_lin_w'].T + params['enc_lin_b']

    vqp = params['vq']
    ze = jax.lax.conv_general_dilated(
        jnp.transpose(e, (0, 2, 1)), vqp['in_w'], window_strides=(1,),
        padding=[(0, 0)], dimension_numbers=('NCH', 'OIH', 'NCH'))
    ze = ze + vqp['in_b'][None, :, None]
    ze_flat = jnp.transpose(ze, (0, 2, 1)).reshape(B * TH, CD)
    enc_n = _normalize_ref(ze_flat)
    cb_n = _normalize_ref(vqp['codebook'])
    dist = (jnp.sum(enc_n ** 2, axis=1, keepdims=True)
            - 2.0 * enc_n @ cb_n.T
            + jnp.sum(cb_n ** 2, axis=1)[None, :])
    idx_flat = jnp.argmax(-dist, axis=1)
    indices = idx_flat.reshape(B, TH)

    zq_flat, loss = _vq_gather(ze_flat, idx_flat, vqp)
    zq = zq_flat.reshape(B, TH, H)

    db = _backbone(zq, params['dec'], H)
    d = _ln_linear(db, params['dec']['final_g'], params['dec']['final_b'],
                   params['dec_lin_w'], params['dec_lin_b'])

    uw = jnp.moveaxis(params['up_w'], 2, 0)
    uws = jnp.stack([uw[0], uw[1] + uw[2], uw[0] + uw[1], uw[2]])
    x_rec = _upsample(d, uws, params['up_b'])

    all_indices = indices[None].astype(jnp.int32)
    return x_rec, loss, all_indices
